# baseline (speedup 1.0000x reference)
import functools

import jax
import jax.numpy as jnp
from jax import lax
from jax.experimental import pallas as pl
from jax.experimental.pallas import tpu as pltpu
from jax.experimental.pallas import tpu_sc as plsc

B = 16384
D = 16
NC = 2
NS = 16
L = 16
NW = NC * NS
BW = B // NW
CH = 128
NCH = BW // CH
R = 16
G = BW // L

_mesh = plsc.VectorSubcoreMesh(core_axis_name="c", subcore_axis_name="s")


@functools.partial(
    pl.kernel,
    out_type=jax.ShapeDtypeStruct((B,), jnp.float32),
    mesh=_mesh,
    compiler_params=pltpu.CompilerParams(
        needs_layout_passes=False, use_tc_tiling_on_sc=True),
    scratch_types=(
        pltpu.VMEM((BW,), jnp.int32),
        pltpu.VMEM((BW,), jnp.int32),
        pltpu.VMEM((D, R * 128), jnp.float32),
        pltpu.VMEM((D, R * 128), jnp.float32),
        pltpu.VMEM((L * D,), jnp.float32),
        pltpu.VMEM((BW,), jnp.float32),
        pltpu.SemaphoreType.DMA((R,)),
    ),
)
def _fm_dot(uid_hbm, iid_hbm, uft_hbm, ift_hbm, dot_hbm,
            uidx_v, iidx_v, uwin_v, iwin_v, prod_v, dot_v, sems):
    wid = lax.axis_index("s") * NC + lax.axis_index("c")
    base = wid * BW

    pltpu.sync_copy(uid_hbm.at[pl.ds(base, BW)], uidx_v)
    pltpu.sync_copy(iid_hbm.at[pl.ds(base, BW)], iidx_v)

    lane = lax.iota(jnp.int32, L)

    def fire(u_scalar, i_scalar, r):
        sl = pl.ds(r * 128, 128)
        us = pl.multiple_of((u_scalar // 128) * 128, 128)
        i_s = pl.multiple_of((i_scalar // 128) * 128, 128)
        pltpu.async_copy(uft_hbm.at[:, pl.ds(us, 128)],
                         uwin_v.at[:, sl], sems.at[r])
        pltpu.async_copy(ift_hbm.at[:, pl.ds(i_s, 128)],
                         iwin_v.at[:, sl], sems.at[r])

    def drain(r):
        sl = pl.ds(r * 128, 128)
        pltpu.make_async_copy(uft_hbm.at[:, pl.ds(0, 128)],
                              uwin_v.at[:, sl], sems.at[r]).wait()
        pltpu.make_async_copy(ift_hbm.at[:, pl.ds(0, 128)],
                              iwin_v.at[:, sl], sems.at[r]).wait()

    uvec0 = uidx_v[pl.ds(0, L)]
    ivec0 = iidx_v[pl.ds(0, L)]
    for r in range(R):
        fire(uvec0[r], ivec0[r], r)

    def group(g, carry):
        uvec = uidx_v[pl.ds(g * L, L)]
        ivec = iidx_v[pl.ds(g * L, L)]
        nbase = jnp.minimum((g + 1) * L, BW - L)
        uvec_n = uidx_v[pl.ds(nbase, L)]
        ivec_n = iidx_v[pl.ds(nbase, L)]
        for j in range(L):
            r = j % R
            drain(r)
            ucol = jnp.full((L,), r * 128, jnp.int32) + (uvec[j] % 128)
            icol = jnp.full((L,), r * 128, jnp.int32) + (ivec[j] % 128)
            u16 = plsc.load_gather(uwin_v, [lane, ucol])
            i16 = plsc.load_gather(iwin_v, [lane, icol])
            prod_v[pl.ds(j * D, D)] = u16 * i16
            if j + R < L:
                fire(uvec[j + R], ivec[j + R], r)
            else:
                fire(uvec_n[j + R - L], ivec_n[j + R - L], r)
        acc = jnp.zeros((L,), jnp.float32)
        for k in range(D):
            acc = acc + plsc.load_gather(prod_v, [lane * D + k])
        dot_v[pl.ds(g * L, L)] = acc
        return carry

    lax.fori_loop(0, G, group, 0)

    for r in range(R):
        drain(r)

    pltpu.sync_copy(dot_v, dot_hbm.at[pl.ds(base, BW)])


@functools.partial(
    pl.kernel,
    out_type=(
        jax.ShapeDtypeStruct((B,), jnp.float32),
        jax.ShapeDtypeStruct((B,), jnp.float32),
    ),
    mesh=_mesh,
    compiler_params=pltpu.CompilerParams(
        needs_layout_passes=False, use_tc_tiling_on_sc=False),
    scratch_types=(
        pltpu.VMEM((NCH, CH), jnp.int32),
        pltpu.VMEM((NCH, CH), jnp.int32),
        pltpu.VMEM((BW,), jnp.float32),
        pltpu.VMEM((BW,), jnp.float32),
        pltpu.VMEM((L,), jnp.float32),
        pltpu.VMEM((BW,), jnp.float32),
        pltpu.VMEM((BW,), jnp.float32),
        pltpu.VMEM((BW,), jnp.float32),
        pltpu.SemaphoreType.DMA,
    ),
)
def _fm_bias(uid_hbm, iid_hbm, ub_hbm, ib_hbm, gb_hbm, dot_hbm,
             pred_hbm, cvr_hbm,
             uidx_v, iidx_v, ubias_v, ibias_v, gb_v, dot_v,
             pred_v, cvr_v, sem):
    wid = lax.axis_index("s") * NC + lax.axis_index("c")
    base = wid * BW

    pltpu.sync_copy(uid_hbm.at[pl.ds(wid * NCH, NCH)], uidx_v)
    pltpu.sync_copy(iid_hbm.at[pl.ds(wid * NCH, NCH)], iidx_v)
    pltpu.sync_copy(gb_hbm, gb_v)
    pltpu.sync_copy(dot_hbm.at[pl.ds(base, BW)], dot_v)

    copies = []
    for j in range(NCH):
        sl = pl.ds(j * CH, CH)
        copies.append(
            pltpu.async_copy(ub_hbm.at[uidx_v.at[j]], ubias_v.at[sl], sem))
        copies.append(
            pltpu.async_copy(ib_hbm.at[iidx_v.at[j]], ibias_v.at[sl], sem))
    for c in copies:
        c.wait()

    gb = gb_v[...]

    def group(g, carry):
        sl = pl.ds(g * L, L)
        p = dot_v[sl] + ubias_v[sl] + ibias_v[sl] + gb
        pred_v[sl] = p
        cvr_v[sl] = 1.0 / (1.0 + jnp.exp(-p))
        return carry

    lax.fori_loop(0, G, group, 0)

    pltpu.sync_copy(pred_v, pred_hbm.at[pl.ds(base, BW)])
    pltpu.sync_copy(cvr_v, cvr_hbm.at[pl.ds(base, BW)])


def kernel(user_id, item_id, user_factors, item_factors, user_bias,
           item_bias, global_bias):
    uid1 = jnp.asarray(user_id, jnp.int32)
    iid1 = jnp.asarray(item_id, jnp.int32)
    uid2 = uid1.reshape(NW * NCH, CH)
    iid2 = iid1.reshape(NW * NCH, CH)
    gb = jnp.broadcast_to(jnp.asarray(global_bias, jnp.float32), (L,))
    dot = _fm_dot(uid1, iid1, user_factors.T, item_factors.T)
    pred, cvr = _fm_bias(uid2, iid2, user_bias, item_bias, gb, dot)
    return (pred, cvr)


# speedup vs baseline: 1.0692x; 1.0692x over previous
import functools

import jax
import jax.numpy as jnp
from jax import lax
from jax.experimental import pallas as pl
from jax.experimental.pallas import tpu as pltpu
from jax.experimental.pallas import tpu_sc as plsc

B = 16384
D = 16
NC = 2
NS = 16
L = 16
NW = NC * NS
BW = B // NW
CH = 128
NCH = BW // CH
R = 8
G = BW // L

_mesh = plsc.VectorSubcoreMesh(core_axis_name="c", subcore_axis_name="s")


@functools.partial(
    pl.kernel,
    out_type=(
        jax.ShapeDtypeStruct((B,), jnp.float32),
        jax.ShapeDtypeStruct((B,), jnp.float32),
    ),
    mesh=_mesh,
    compiler_params=pltpu.CompilerParams(
        needs_layout_passes=False, use_tc_tiling_on_sc=True),
    scratch_types=(
        pltpu.VMEM((BW,), jnp.int32),
        pltpu.VMEM((BW,), jnp.int32),
        pltpu.VMEM((D, R * 128), jnp.float32),
        pltpu.VMEM((D, R * 128), jnp.float32),
        pltpu.VMEM((L * D,), jnp.float32),
        pltpu.VMEM((BW,), jnp.float32),
        pltpu.VMEM((BW,), jnp.float32),
        pltpu.VMEM((L,), jnp.float32),
        pltpu.VMEM((BW,), jnp.float32),
        pltpu.VMEM((BW,), jnp.float32),
        pltpu.SemaphoreType.DMA((R,)),
        pltpu.SemaphoreType.DMA,
    ),
)
def _fm_sc(uid_hbm, iid_hbm, uft_hbm, ift_hbm, ub_hbm, ib_hbm, gb_hbm,
           pred_hbm, cvr_hbm,
           uidx_v, iidx_v, uwin_v, iwin_v, prod_v, ubias_v, ibias_v, gb_v,
           pred_v, cvr_v, sems, bsem):
    wid = lax.axis_index("s") * NC + lax.axis_index("c")
    base = wid * BW

    pltpu.sync_copy(uid_hbm.at[pl.ds(base, BW)], uidx_v)
    pltpu.sync_copy(iid_hbm.at[pl.ds(base, BW)], iidx_v)
    pltpu.sync_copy(gb_hbm, gb_v)

    bias_copies = []
    for j in range(NCH):
        sl = pl.ds(j * CH, CH)
        bias_copies.append(
            pltpu.async_copy(ub_hbm.at[uidx_v.at[pl.ds(j * CH, CH)]],
                             ubias_v.at[sl], bsem))
        bias_copies.append(
            pltpu.async_copy(ib_hbm.at[iidx_v.at[pl.ds(j * CH, CH)]],
                             ibias_v.at[sl], bsem))

    lane = lax.iota(jnp.int32, L)

    def fire(u_scalar, i_scalar, r):
        sl = pl.ds(r * 128, 128)
        us = pl.multiple_of((u_scalar // 128) * 128, 128)
        i_s = pl.multiple_of((i_scalar // 128) * 128, 128)
        pltpu.async_copy(uft_hbm.at[:, pl.ds(us, 128)],
                         uwin_v.at[:, sl], sems.at[r])
        pltpu.async_copy(ift_hbm.at[:, pl.ds(i_s, 128)],
                         iwin_v.at[:, sl], sems.at[r])

    def drain(r):
        sl = pl.ds(r * 128, 128)
        pltpu.make_async_copy(uft_hbm.at[:, pl.ds(0, 128)],
                              uwin_v.at[:, sl], sems.at[r]).wait()
        pltpu.make_async_copy(ift_hbm.at[:, pl.ds(0, 128)],
                              iwin_v.at[:, sl], sems.at[r]).wait()

    uvec0 = uidx_v[pl.ds(0, L)]
    ivec0 = iidx_v[pl.ds(0, L)]
    for r in range(R):
        fire(uvec0[r], ivec0[r], r)

    def group(g, carry):
        uvec = uidx_v[pl.ds(g * L, L)]
        ivec = iidx_v[pl.ds(g * L, L)]
        nbase = jnp.minimum((g + 1) * L, BW - L)
        uvec_n = uidx_v[pl.ds(nbase, L)]
        ivec_n = iidx_v[pl.ds(nbase, L)]
        for j in range(L):
            r = j % R
            drain(r)
            ucol = jnp.full((L,), r * 128, jnp.int32) + (uvec[j] % 128)
            icol = jnp.full((L,), r * 128, jnp.int32) + (ivec[j] % 128)
            u16 = plsc.load_gather(uwin_v, [lane, ucol])
            i16 = plsc.load_gather(iwin_v, [lane, icol])
            prod_v[pl.ds(j * D, D)] = u16 * i16
            if j + R < L:
                fire(uvec[j + R], ivec[j + R], r)
            else:
                fire(uvec_n[j + R - L], ivec_n[j + R - L], r)
        acc = jnp.zeros((L,), jnp.float32)
        for k in range(D):
            acc = acc + plsc.load_gather(prod_v, [lane * D + k])
        pred_v[pl.ds(g * L, L)] = acc
        return carry

    lax.fori_loop(0, G, group, 0)

    for r in range(R):
        drain(r)
    for c in bias_copies:
        c.wait()

    gb = gb_v[...]

    def finish(g, carry):
        sl = pl.ds(g * L, L)
        p = pred_v[sl] + ubias_v[sl] + ibias_v[sl] + gb
        pred_v[sl] = p
        cvr_v[sl] = 1.0 / (1.0 + jnp.exp(-p))
        return carry

    lax.fori_loop(0, G, finish, 0)

    pltpu.sync_copy(pred_v, pred_hbm.at[pl.ds(base, BW)])
    pltpu.sync_copy(cvr_v, cvr_hbm.at[pl.ds(base, BW)])


def kernel(user_id, item_id, user_factors, item_factors, user_bias,
           item_bias, global_bias):
    uid1 = jnp.asarray(user_id, jnp.int32)
    iid1 = jnp.asarray(item_id, jnp.int32)
    gb = jnp.broadcast_to(jnp.asarray(global_bias, jnp.float32), (L,))
    pred, cvr = _fm_sc(uid1, iid1, user_factors.T, item_factors.T,
                       user_bias, item_bias, gb)
    return (pred, cvr)


# speedup vs baseline: 1.0694x; 1.0002x over previous
import functools

import jax
import jax.numpy as jnp
from jax import lax
from jax.experimental import pallas as pl
from jax.experimental.pallas import tpu as pltpu
from jax.experimental.pallas import tpu_sc as plsc

B = 16384
D = 16
NC = 2
NS = 16
L = 16
NW = NC * NS
BW = B // NW
CH = 128
NCH = BW // CH
R = 8
G = BW // L

_mesh = plsc.VectorSubcoreMesh(core_axis_name="c", subcore_axis_name="s")


@functools.partial(
    pl.kernel,
    out_type=(
        jax.ShapeDtypeStruct((B,), jnp.float32),
        jax.ShapeDtypeStruct((B,), jnp.float32),
    ),
    mesh=_mesh,
    compiler_params=pltpu.CompilerParams(
        needs_layout_passes=False, use_tc_tiling_on_sc=True),
    scratch_types=(
        pltpu.VMEM((BW,), jnp.int32),
        pltpu.VMEM((BW,), jnp.int32),
        pltpu.VMEM((D, R * 256), jnp.float32),
        pltpu.VMEM((L * D,), jnp.float32),
        pltpu.VMEM((BW,), jnp.float32),
        pltpu.VMEM((BW,), jnp.float32),
        pltpu.VMEM((L,), jnp.float32),
        pltpu.VMEM((BW,), jnp.float32),
        pltpu.VMEM((BW,), jnp.float32),
        pltpu.SemaphoreType.DMA((R,)),
        pltpu.SemaphoreType.DMA,
    ),
)
def _fm_sc(uid_hbm, iid_hbm, uft_hbm, ift_hbm, ub_hbm, ib_hbm, gb_hbm,
           pred_hbm, cvr_hbm,
           uidx_v, iidx_v, win_v, prod_v, ubias_v, ibias_v, gb_v,
           pred_v, cvr_v, sems, bsem):
    wid = lax.axis_index("s") * NC + lax.axis_index("c")
    base = wid * BW

    pltpu.sync_copy(uid_hbm.at[pl.ds(base, BW)], uidx_v)
    pltpu.sync_copy(iid_hbm.at[pl.ds(base, BW)], iidx_v)
    pltpu.sync_copy(gb_hbm, gb_v)

    bias_copies = []
    for j in range(NCH):
        sl = pl.ds(j * CH, CH)
        bias_copies.append(
            pltpu.async_copy(ub_hbm.at[uidx_v.at[pl.ds(j * CH, CH)]],
                             ubias_v.at[sl], bsem))
        bias_copies.append(
            pltpu.async_copy(ib_hbm.at[iidx_v.at[pl.ds(j * CH, CH)]],
                             ibias_v.at[sl], bsem))

    lane = lax.iota(jnp.int32, L)

    def fire(u_scalar, i_scalar, r):
        us = pl.multiple_of((u_scalar // 128) * 128, 128)
        i_s = pl.multiple_of((i_scalar // 128) * 128, 128)
        pltpu.async_copy(uft_hbm.at[:, pl.ds(us, 128)],
                         win_v.at[:, pl.ds(r * 256, 128)], sems.at[r])
        pltpu.async_copy(ift_hbm.at[:, pl.ds(i_s, 128)],
                         win_v.at[:, pl.ds(r * 256 + 128, 128)], sems.at[r])

    def drain(r):
        pltpu.make_async_copy(uft_hbm.at[:, pl.ds(0, 256)],
                              win_v.at[:, pl.ds(r * 256, 256)],
                              sems.at[r]).wait()

    uvec0 = uidx_v[pl.ds(0, L)]
    ivec0 = iidx_v[pl.ds(0, L)]
    for r in range(R):
        fire(uvec0[r], ivec0[r], r)

    def group(g, carry):
        uvec = uidx_v[pl.ds(g * L, L)]
        ivec = iidx_v[pl.ds(g * L, L)]
        nbase = jnp.minimum((g + 1) * L, BW - L)
        uvec_n = uidx_v[pl.ds(nbase, L)]
        ivec_n = iidx_v[pl.ds(nbase, L)]
        for j in range(L):
            r = j % R
            drain(r)
            ucol = jnp.full((L,), r * 256, jnp.int32) + (uvec[j] % 128)
            icol = jnp.full((L,), r * 256 + 128, jnp.int32) + (ivec[j] % 128)
            u16 = plsc.load_gather(win_v, [lane, ucol])
            i16 = plsc.load_gather(win_v, [lane, icol])
            prod_v[pl.ds(j * D, D)] = u16 * i16
            if j + R < L:
                fire(uvec[j + R], ivec[j + R], r)
            else:
                fire(uvec_n[j + R - L], ivec_n[j + R - L], r)
        acc = jnp.zeros((L,), jnp.float32)
        for k in range(D):
            acc = acc + plsc.load_gather(prod_v, [lane * D + k])
        pred_v[pl.ds(g * L, L)] = acc
        return carry

    lax.fori_loop(0, G, group, 0)

    for r in range(R):
        drain(r)
    for c in bias_copies:
        c.wait()

    gb = gb_v[...]

    def finish(g, carry):
        sl = pl.ds(g * L, L)
        p = pred_v[sl] + ubias_v[sl] + ibias_v[sl] + gb
        pred_v[sl] = p
        cvr_v[sl] = 1.0 / (1.0 + jnp.exp(-p))
        return carry

    lax.fori_loop(0, G, finish, 0)

    pltpu.sync_copy(pred_v, pred_hbm.at[pl.ds(base, BW)])
    pltpu.sync_copy(cvr_v, cvr_hbm.at[pl.ds(base, BW)])


def kernel(user_id, item_id, user_factors, item_factors, user_bias,
           item_bias, global_bias):
    uid1 = jnp.asarray(user_id, jnp.int32)
    iid1 = jnp.asarray(item_id, jnp.int32)
    gb = jnp.broadcast_to(jnp.asarray(global_bias, jnp.float32), (L,))
    pred, cvr = _fm_sc(uid1, iid1, user_factors.T, item_factors.T,
                       user_bias, item_bias, gb)
    return (pred, cvr)
